# baseline (speedup 1.0000x reference)
import functools

import jax
import jax.numpy as jnp
from jax import lax
from jax.experimental import pallas as pl
from jax.experimental.pallas import tpu as pltpu
from jax.experimental.pallas import tpu_sc as plsc

B = 16384
D = 50
H = 128
NC = 2
NS = 16
NW = NC * NS
BPW = B // NW
CHUNK = 128
NCHUNK = BPW // CHUNK


def _gather_body(uidx_hbm, midx_hbm, ut_hbm, mt_hbm, ue_hbm, me_hbm,
                 uidx_v, midx_v, urows_v, mrows_v, sem_u, sem_m):
    wid = lax.axis_index("s") * NC + lax.axis_index("c")
    base = wid * BPW
    pltpu.sync_copy(uidx_hbm.at[wid], uidx_v)
    pltpu.sync_copy(midx_hbm.at[wid], midx_v)
    ucopies = []
    mcopies = []
    for c in range(NCHUNK):
        cu = pltpu.make_async_copy(
            ut_hbm.at[uidx_v.at[c]], urows_v.at[pl.ds(c * CHUNK, CHUNK)], sem_u)
        cu.start()
        ucopies.append(cu)
        cm = pltpu.make_async_copy(
            mt_hbm.at[midx_v.at[c]], mrows_v.at[pl.ds(c * CHUNK, CHUNK)], sem_m)
        cm.start()
        mcopies.append(cm)
    for cu in ucopies:
        cu.wait()
    for cm in mcopies:
        cm.wait()
    pltpu.sync_copy(urows_v, ue_hbm.at[pl.ds(base, BPW)])
    pltpu.sync_copy(mrows_v, me_hbm.at[pl.ds(base, BPW)])


def _sc_gather(uidx, midx, ut, mt):
    mesh = plsc.VectorSubcoreMesh(core_axis_name="c", subcore_axis_name="s")
    fn = pl.kernel(
        _gather_body,
        out_type=[
            jax.ShapeDtypeStruct((B, D), jnp.float32),
            jax.ShapeDtypeStruct((B, D), jnp.float32),
        ],
        mesh=mesh,
        scratch_types=[
            pltpu.VMEM((NCHUNK, CHUNK), jnp.int32),
            pltpu.VMEM((NCHUNK, CHUNK), jnp.int32),
            pltpu.VMEM((BPW, D), jnp.float32),
            pltpu.VMEM((BPW, D), jnp.float32),
            pltpu.SemaphoreType.DMA,
            pltpu.SemaphoreType.DMA,
        ],
        compiler_params=pltpu.CompilerParams(use_tc_tiling_on_sc=False),
    )
    return fn(uidx, midx, ut, mt)


def _mlp_body(ue_ref, me_ref, w1a_ref, w1b_ref, b1_ref, w2_ref, b2_ref, out_ref):
    h = jnp.dot(ue_ref[...], w1a_ref[...], preferred_element_type=jnp.float32)
    h = h + jnp.dot(me_ref[...], w1b_ref[...], preferred_element_type=jnp.float32)
    h = jnp.maximum(h + b1_ref[...], 0.0)
    out = jnp.sum(h * w2_ref[...], axis=1, keepdims=True) + b2_ref[...]
    out_ref[...] = out


def _tc_mlp(ue, me, W1, b1, W2, b2):
    w1a = W1[:D]
    w1b = W1[D:]
    b1r = b1.reshape(1, H)
    w2r = W2.reshape(1, H)
    b2r = b2.reshape(1, 1)
    nblk = 8
    bb = B // nblk
    out = pl.pallas_call(
        _mlp_body,
        grid=(nblk,),
        in_specs=[
            pl.BlockSpec((bb, D), lambda i: (i, 0)),
            pl.BlockSpec((bb, D), lambda i: (i, 0)),
            pl.BlockSpec((D, H), lambda i: (0, 0)),
            pl.BlockSpec((D, H), lambda i: (0, 0)),
            pl.BlockSpec((1, H), lambda i: (0, 0)),
            pl.BlockSpec((1, H), lambda i: (0, 0)),
            pl.BlockSpec((1, 1), lambda i: (0, 0)),
        ],
        out_specs=pl.BlockSpec((bb, 1), lambda i: (i, 0)),
        out_shape=jax.ShapeDtypeStruct((B, 1), jnp.float32),
    )(ue, me, w1a, w1b, b1r, w2r, b2r)
    return out[:, 0]


def kernel(user, movie, user_table, movie_table, W1, b1, W2, b2):
    uidx = user.astype(jnp.int32).reshape(NW, NCHUNK, CHUNK)
    midx = movie.astype(jnp.int32).reshape(NW, NCHUNK, CHUNK)
    ue, me = _sc_gather(uidx, midx, user_table, movie_table)
    return _tc_mlp(ue, me, W1, b1, W2, b2)


# speedup vs baseline: 3.6697x; 3.6697x over previous
import functools

import jax
import jax.numpy as jnp
from jax import lax
from jax.experimental import pallas as pl
from jax.experimental.pallas import tpu as pltpu
from jax.experimental.pallas import tpu_sc as plsc

B = 16384
D = 50
H = 128
NC = 2
NS = 16
NW = NC * NS
BPW = B // NW
CHUNK = 128
NCHUNK = BPW // CHUNK


NBUF = 8
STAGE = 128


def _gather_body(uidx_hbm, midx_hbm, ut_hbm, mt_hbm, ue_hbm, me_hbm,
                 uidx_v, midx_v, urows_v, mrows_v, usem, msem):
    wid = lax.axis_index("s") * NC + lax.axis_index("c")
    base = wid * BPW
    pltpu.sync_copy(uidx_hbm.at[pl.ds(base, BPW)], uidx_v.at[pl.ds(0, BPW)])
    pltpu.sync_copy(midx_hbm.at[pl.ds(base, BPW)], midx_v.at[pl.ds(0, BPW)])

    def fire(i, row, slot):
        ui = uidx_v[pl.ds(i, 16)][0]
        mi = midx_v[pl.ds(i, 16)][0]
        pltpu.make_async_copy(
            ut_hbm.at[ui], urows_v.at[row], usem.at[slot]).start()
        pltpu.make_async_copy(
            mt_hbm.at[mi], mrows_v.at[row], msem.at[slot]).start()

    def drain(slot):
        pltpu.make_async_copy(
            ut_hbm.at[0], urows_v.at[0], usem.at[slot]).wait()
        pltpu.make_async_copy(
            mt_hbm.at[0], mrows_v.at[0], msem.at[slot]).wait()

    for st in range(BPW // STAGE):
        st0 = st * STAGE

        for i in range(NBUF):
            fire(st0 + i, i, i)

        def step(i, carry):
            slot = lax.rem(i, NBUF)
            drain(slot)

            @pl.when(i + NBUF < STAGE)
            def _():
                fire(st0 + i + NBUF, i + NBUF, slot)

            return carry

        lax.fori_loop(0, STAGE, step, 0)
        pltpu.sync_copy(urows_v, ue_hbm.at[pl.ds(base + st0, STAGE)])
        pltpu.sync_copy(mrows_v, me_hbm.at[pl.ds(base + st0, STAGE)])


def _sc_gather(uidx, midx, ut, mt):
    mesh = plsc.VectorSubcoreMesh(core_axis_name="c", subcore_axis_name="s")
    fn = pl.kernel(
        _gather_body,
        out_type=[
            jax.ShapeDtypeStruct((B, D), jnp.float32),
            jax.ShapeDtypeStruct((B, D), jnp.float32),
        ],
        mesh=mesh,
        scratch_types=[
            pltpu.VMEM((BPW + 16,), jnp.int32),
            pltpu.VMEM((BPW + 16,), jnp.int32),
            pltpu.VMEM((STAGE, D), jnp.float32),
            pltpu.VMEM((STAGE, D), jnp.float32),
            pltpu.SemaphoreType.DMA((NBUF,)),
            pltpu.SemaphoreType.DMA((NBUF,)),
        ],
    )
    return fn(uidx, midx, ut, mt)


def _mlp_body(ue_ref, me_ref, w1a_ref, w1b_ref, b1_ref, w2_ref, b2_ref, out_ref):
    h = jnp.dot(ue_ref[...], w1a_ref[...], preferred_element_type=jnp.float32)
    h = h + jnp.dot(me_ref[...], w1b_ref[...], preferred_element_type=jnp.float32)
    h = jnp.maximum(h + b1_ref[...], 0.0)
    out = jnp.sum(h * w2_ref[...], axis=1, keepdims=True) + b2_ref[...]
    out_ref[...] = out


def _tc_mlp(ue, me, W1, b1, W2, b2):
    w1a = W1[:D]
    w1b = W1[D:]
    b1r = b1.reshape(1, H)
    w2r = W2.reshape(1, H)
    b2r = b2.reshape(1, 1)
    nblk = 8
    bb = B // nblk
    out = pl.pallas_call(
        _mlp_body,
        grid=(nblk,),
        in_specs=[
            pl.BlockSpec((bb, D), lambda i: (i, 0)),
            pl.BlockSpec((bb, D), lambda i: (i, 0)),
            pl.BlockSpec((D, H), lambda i: (0, 0)),
            pl.BlockSpec((D, H), lambda i: (0, 0)),
            pl.BlockSpec((1, H), lambda i: (0, 0)),
            pl.BlockSpec((1, H), lambda i: (0, 0)),
            pl.BlockSpec((1, 1), lambda i: (0, 0)),
        ],
        out_specs=pl.BlockSpec((bb, 1), lambda i: (i, 0)),
        out_shape=jax.ShapeDtypeStruct((B, 1), jnp.float32),
    )(ue, me, w1a, w1b, b1r, w2r, b2r)
    return out[:, 0]


def kernel(user, movie, user_table, movie_table, W1, b1, W2, b2):
    uidx = user.astype(jnp.int32)
    midx = movie.astype(jnp.int32)
    ue, me = _sc_gather(uidx, midx, user_table, movie_table)
    return _tc_mlp(ue, me, W1, b1, W2, b2)


# speedup vs baseline: 4.1324x; 1.1261x over previous
import functools

import jax
import jax.numpy as jnp
from jax import lax
from jax.experimental import pallas as pl
from jax.experimental.pallas import tpu as pltpu
from jax.experimental.pallas import tpu_sc as plsc

B = 16384
D = 50
H = 128
NC = 2
NS = 16
NW = NC * NS
BPW = B // NW
STAGE = 128
GRP = 16
NGRP = STAGE // GRP


def _gather_body(uidx_hbm, midx_hbm, ut_hbm, mt_hbm, ue_hbm, me_hbm,
                 uidx_v, midx_v, urows_v, mrows_v, sem):
    wid = lax.axis_index("s") * NC + lax.axis_index("c")
    base = wid * BPW
    pltpu.sync_copy(uidx_hbm.at[pl.ds(base, BPW)], uidx_v)
    pltpu.sync_copy(midx_hbm.at[pl.ds(base, BPW)], midx_v)

    for st in range(BPW // STAGE):
        st0 = st * STAGE

        def fire(g, slot):
            uv = uidx_v[pl.ds(st0 + g * GRP, GRP)]
            mv = midx_v[pl.ds(st0 + g * GRP, GRP)]
            for j in range(GRP):
                pltpu.make_async_copy(
                    ut_hbm.at[uv[j]], urows_v.at[g * GRP + j],
                    sem.at[slot]).start()
                pltpu.make_async_copy(
                    mt_hbm.at[mv[j]], mrows_v.at[g * GRP + j],
                    sem.at[slot]).start()

        def drain(slot):
            for j in range(GRP):
                pltpu.make_async_copy(
                    ut_hbm.at[0], urows_v.at[0], sem.at[slot]).wait()
                pltpu.make_async_copy(
                    mt_hbm.at[0], mrows_v.at[0], sem.at[slot]).wait()

        fire(0, 0)
        fire(1, 1)

        def stepg(g, carry):
            sb = lax.rem(g, 2)
            drain(sb)

            @pl.when(g + 2 < NGRP)
            def _():
                fire(g + 2, sb)

            return carry

        lax.fori_loop(0, NGRP, stepg, 0)
        pltpu.sync_copy(urows_v, ue_hbm.at[pl.ds(base + st0, STAGE)])
        pltpu.sync_copy(mrows_v, me_hbm.at[pl.ds(base + st0, STAGE)])


def _sc_gather(uidx, midx, ut, mt):
    mesh = plsc.VectorSubcoreMesh(core_axis_name="c", subcore_axis_name="s")
    fn = pl.kernel(
        _gather_body,
        out_type=[
            jax.ShapeDtypeStruct((B, D), jnp.float32),
            jax.ShapeDtypeStruct((B, D), jnp.float32),
        ],
        mesh=mesh,
        scratch_types=[
            pltpu.VMEM((BPW,), jnp.int32),
            pltpu.VMEM((BPW,), jnp.int32),
            pltpu.VMEM((STAGE, D), jnp.float32),
            pltpu.VMEM((STAGE, D), jnp.float32),
            pltpu.SemaphoreType.DMA((2,)),
        ],
    )
    return fn(uidx, midx, ut, mt)


def _mlp_body(ue_ref, me_ref, w1a_ref, w1b_ref, b1_ref, w2_ref, b2_ref, out_ref):
    h = jnp.dot(ue_ref[...], w1a_ref[...], preferred_element_type=jnp.float32)
    h = h + jnp.dot(me_ref[...], w1b_ref[...], preferred_element_type=jnp.float32)
    h = jnp.maximum(h + b1_ref[...], 0.0)
    out = jnp.sum(h * w2_ref[...], axis=1, keepdims=True) + b2_ref[...]
    out_ref[...] = out


def _tc_mlp(ue, me, W1, b1, W2, b2):
    w1a = W1[:D]
    w1b = W1[D:]
    b1r = b1.reshape(1, H)
    w2r = W2.reshape(1, H)
    b2r = b2.reshape(1, 1)
    nblk = 8
    bb = B // nblk
    out = pl.pallas_call(
        _mlp_body,
        grid=(nblk,),
        in_specs=[
            pl.BlockSpec((bb, D), lambda i: (i, 0)),
            pl.BlockSpec((bb, D), lambda i: (i, 0)),
            pl.BlockSpec((D, H), lambda i: (0, 0)),
            pl.BlockSpec((D, H), lambda i: (0, 0)),
            pl.BlockSpec((1, H), lambda i: (0, 0)),
            pl.BlockSpec((1, H), lambda i: (0, 0)),
            pl.BlockSpec((1, 1), lambda i: (0, 0)),
        ],
        out_specs=pl.BlockSpec((bb, 1), lambda i: (i, 0)),
        out_shape=jax.ShapeDtypeStruct((B, 1), jnp.float32),
    )(ue, me, w1a, w1b, b1r, w2r, b2r)
    return out[:, 0]


def kernel(user, movie, user_table, movie_table, W1, b1, W2, b2):
    uidx = user.astype(jnp.int32)
    midx = movie.astype(jnp.int32)
    ue, me = _sc_gather(uidx, midx, user_table, movie_table)
    return _tc_mlp(ue, me, W1, b1, W2, b2)
